# Initial kernel scaffold; baseline (speedup 1.0000x reference)
import jax
import jax.numpy as jnp
from jax.experimental import pallas as pl


def kernel(h, edge_index, snorm_n, snorm_e, W1, a1, W2, a2, mask_train, mask_fixed):
    raise NotImplementedError("write your pallas kernel here")


# speedup vs baseline: 13.4852x; 13.4852x over previous
import functools

import jax
import jax.numpy as jnp
from jax import lax
from jax.experimental import pallas as pl
from jax.experimental.pallas import tpu as pltpu
from jax.experimental.pallas import tpu_sc as plsc

NC = 2
NS = 16
NW = NC * NS
K = 128


@functools.lru_cache(maxsize=None)
def _tc_linear(n, d_in, d_out):
    def body(x_ref, w_ref, a_ref, zaug_ref, er_ref):
        z = lax.dot_general(x_ref[...], w_ref[...], (((1,), (1,)), ((), ())),
                            preferred_element_type=jnp.float32)
        al = a_ref[0, :d_out].reshape(d_out, 1)
        ar = a_ref[0, d_out:].reshape(d_out, 1)
        el = jnp.dot(z, al, preferred_element_type=jnp.float32)
        er = jnp.dot(z, ar, preferred_element_type=jnp.float32)
        pad = jnp.zeros((n, 15), jnp.float32)
        zaug_ref[...] = jnp.concatenate([z, el, pad], axis=1)
        er_ref[...] = jnp.concatenate([er, pad], axis=1)

    return pl.pallas_call(
        body,
        out_shape=[
            jax.ShapeDtypeStruct((n, d_out + 16), jnp.float32),
            jax.ShapeDtypeStruct((n, 16), jnp.float32),
        ],
    )


@functools.lru_cache(maxsize=None)
def _tc_combine(n, n_pad, d_in, d_out):
    def body(p_ref, w_ref, a_ref, zaug_ref, er_ref):
        ps = p_ref[0, :n, :] + p_ref[1, :n, :]
        h1 = ps[:, :d_in] / ps[:, d_in:d_in + 1]
        z = lax.dot_general(h1, w_ref[...], (((1,), (1,)), ((), ())),
                            preferred_element_type=jnp.float32)
        al = a_ref[0, :d_out].reshape(d_out, 1)
        ar = a_ref[0, d_out:].reshape(d_out, 1)
        el = jnp.dot(z, al, preferred_element_type=jnp.float32)
        er = jnp.dot(z, ar, preferred_element_type=jnp.float32)
        pad = jnp.zeros((n, 15), jnp.float32)
        zaug_ref[...] = jnp.concatenate([z, el, pad], axis=1)
        er_ref[...] = jnp.concatenate([er, pad], axis=1)

    return pl.pallas_call(
        body,
        out_shape=[
            jax.ShapeDtypeStruct((n, d_out + 16), jnp.float32),
            jax.ShapeDtypeStruct((n, 16), jnp.float32),
        ],
    )


@functools.lru_cache(maxsize=None)
def _tc_finalize(n, n_pad, d):
    def body(p_ref, o_ref):
        ps = p_ref[0, :n, :] + p_ref[1, :n, :]
        o_ref[...] = ps[:, :d] / ps[:, d:d + 1]

    return pl.pallas_call(
        body, out_shape=jax.ShapeDtypeStruct((n, d), jnp.float32))


@functools.lru_cache(maxsize=None)
def _sc_layer(n, n_pad, nb, d, e_total):
    aug = d + 16
    rows_per_tile = n_pad // NS
    zb = rows_per_tile // K
    mesh = plsc.VectorSubcoreMesh(core_axis_name="c", subcore_axis_name="s")

    @functools.partial(
        pl.kernel,
        out_type=jax.ShapeDtypeStruct((NC, n_pad, aug), jnp.float32),
        mesh=mesh,
        scratch_types=[
            pltpu.VMEM((K,), jnp.int32),
            pltpu.VMEM((K,), jnp.int32),
            pltpu.VMEM((K,), jnp.float32),
            pltpu.VMEM((K,), jnp.float32),
            pltpu.VMEM((K,), jnp.float32),
            pltpu.VMEM((K, aug), jnp.float32),
            pltpu.VMEM((K, 16), jnp.float32),
            pltpu.VMEM_SHARED((n_pad, aug), jnp.float32),
            pltpu.SemaphoreType.DMA,
            pltpu.SemaphoreType.DMA,
        ],
        compiler_params=pltpu.CompilerParams(use_tc_tiling_on_sc=False,
                                             needs_layout_passes=False),
    )
    def sc_fn(src_h, dst_h, mt_h, mf_h, zaug_h, er_h, p_out,
              src_b, dst_b, mt_b, mf_b, w_v, rows_v, erows_v, accum,
              gsem, esem):
        c = lax.axis_index("c")
        s = lax.axis_index("s")
        wid = s * NC + c

        def zrow(r, carry):
            for q in range(aug // 16):
                rows_v[r, pl.ds(q * 16, 16)] = jnp.zeros((16,), jnp.float32)
            return carry
        lax.fori_loop(0, K, zrow, None)
        for b in range(zb):
            pltpu.sync_copy(rows_v,
                            accum.at[pl.ds(s * rows_per_tile + b * K, K)])
        plsc.subcore_barrier()

        lane = lax.broadcasted_iota(jnp.int32, (16,), 0)
        cd = jnp.full((16,), d, jnp.int32)
        c0 = jnp.zeros((16,), jnp.int32)

        def block(j, carry):
            base = (wid * nb + j) * K
            pltpu.sync_copy(src_h.at[wid, j], src_b)
            pltpu.sync_copy(dst_h.at[wid, j], dst_b)
            pltpu.sync_copy(mt_h.at[wid, j], mt_b)
            pltpu.sync_copy(mf_h.at[wid, j], mf_b)
            cp1 = pltpu.async_copy(zaug_h.at[src_b], rows_v, gsem)
            cp2 = pltpu.async_copy(er_h.at[dst_b], erows_v, esem)
            cp1.wait()
            cp2.wait()

            for q in range(K // 16):
                rvec = lane + q * 16
                ev = plsc.load_gather(rows_v, [rvec, cd]) \
                    + plsc.load_gather(erows_v, [rvec, c0])
                ev = jnp.where(ev >= 0.0, ev, ev * jnp.float32(0.01))
                ev = ev * mt_b[pl.ds(q * 16, 16)] * mf_b[pl.ds(q * 16, 16)]
                w = jnp.exp(ev)
                w = jnp.where(base + q * 16 + lane < e_total, w,
                              jnp.float32(0.0))
                w_v[pl.ds(q * 16, 16)] = w

            def row(r, rcarry):
                wb = plsc.load_gather(w_v, [lane * 0 + r])
                for q in range(d // 16):
                    rows_v[r, pl.ds(q * 16, 16)] = \
                        wb * rows_v[r, pl.ds(q * 16, 16)]
                rows_v[r, pl.ds(d, 16)] = jnp.where(lane == 0, wb,
                                                    jnp.float32(0.0))
                return rcarry
            lax.fori_loop(0, K, row, None)

            pltpu.sync_copy(rows_v, accum.at[dst_b], add=True)
            return carry

        lax.fori_loop(0, nb, block, None)
        plsc.subcore_barrier()

        for b in range(zb):
            off = s * rows_per_tile + b * K
            pltpu.sync_copy(accum.at[pl.ds(off, K)],
                            p_out.at[c, pl.ds(off, K)])

    return sc_fn


def kernel(h, edge_index, snorm_n, snorm_e, W1, a1, W2, a2,
           mask_train, mask_fixed):
    n, d_in = h.shape
    d_hid = W1.shape[0]
    d_out = W2.shape[0]
    e = edge_index.shape[1]
    e_total = e + n
    nb = -(-e_total // (NW * K))
    e_pad = NW * nb * K
    n_pad = -(-n // (NS * K)) * NS * K

    loop = jnp.arange(n, dtype=edge_index.dtype)
    src = jnp.concatenate([edge_index[0], loop])
    dst = jnp.concatenate([edge_index[1], loop])
    pad = (0, e_pad - e_total)
    src_c = jnp.pad(src, pad).reshape(NW, nb, K)
    dst_c = jnp.pad(dst, pad).reshape(NW, nb, K)
    mt_c = jnp.pad(mask_train[:, 0], pad).reshape(NW, nb, K)
    mf_c = jnp.pad(mask_fixed[:, 0], pad).reshape(NW, nb, K)

    zaug1, er1 = _tc_linear(n, d_in, d_hid)(h, W1, a1)
    p1 = _sc_layer(n, n_pad, nb, d_hid, e_total)(
        src_c, dst_c, mt_c, mf_c, zaug1, er1)
    zaug2, er2 = _tc_combine(n, n_pad, d_hid, d_out)(p1, W2, a2)
    p2 = _sc_layer(n, n_pad, nb, d_out, e_total)(
        src_c, dst_c, mt_c, mf_c, zaug2, er2)
    out = _tc_finalize(n, n_pad, d_out)(p2)
    return out[None, :, :]
